# Initial kernel scaffold; baseline (speedup 1.0000x reference)
import jax
import jax.numpy as jnp
from jax.experimental import pallas as pl


def kernel(x, edge_index, W1, W2, Wlink, Wnode):
    raise NotImplementedError("write your pallas kernel here")


# speedup vs baseline: 4.8840x; 4.8840x over previous
import functools

import jax
import jax.numpy as jnp
from jax import lax
from jax.experimental import pallas as pl
from jax.experimental.pallas import tpu as pltpu
from jax.experimental.pallas import tpu_sc as plsc

N = 10000
D = 128
E = 320000
NC = 2
NS = 16
NW = NC * NS
EPR = 128
RPW = 80
E_PAD = NW * RPW * EPR
N_PAD = N + 1
ROWS_PT = 632
NACC = ROWS_PT * NS
DEG_PAD = 640 * NS

_f32 = jnp.float32
_mesh = plsc.VectorSubcoreMesh(core_axis_name="c", subcore_axis_name="s")
_sc_params = pltpu.CompilerParams(needs_layout_passes=False)


def _zero_vmem_2d(ref, nrows):
  def zr(i, _):
    for cc in range(D // 16):
      ref[i, pl.ds(cc * 16, 16)] = jnp.zeros((16,), _f32)
    return 0
  lax.fori_loop(0, nrows, zr, 0)


def _zero_acc_slice(rows, acc, s):
  for j in range(4):
    pltpu.sync_copy(rows, acc.at[pl.ds(s * ROWS_PT + j * EPR, EPR)])
  pltpu.sync_copy(rows.at[pl.ds(0, ROWS_PT - 4 * EPR)],
                  acc.at[pl.ds(s * ROWS_PT + 4 * EPR, ROWS_PT - 4 * EPR)])


def _dump_acc_slice(acc, out_hbm, c, s):
  for j in range(4):
    pltpu.sync_copy(acc.at[pl.ds(s * ROWS_PT + j * EPR, EPR)],
                    out_hbm.at[c, pl.ds(s * ROWS_PT + j * EPR, EPR)])
  pltpu.sync_copy(acc.at[pl.ds(s * ROWS_PT + 4 * EPR, ROWS_PT - 4 * EPR)],
                  out_hbm.at[c, pl.ds(s * ROWS_PT + 4 * EPR, ROWS_PT - 4 * EPR)])


@functools.partial(
    pl.kernel,
    out_type=jax.ShapeDtypeStruct((NC, DEG_PAD), _f32),
    mesh=_mesh,
    compiler_params=_sc_params,
    scratch_types=[
        pltpu.MemorySpace.VMEM((8, EPR), jnp.int32),
        pltpu.MemorySpace.VMEM((640,), _f32),
        pltpu.MemorySpace.VMEM_SHARED((DEG_PAD,), _f32),
    ],
)
def _deg_kernel(dst_hbm, out_hbm, didx, buf, dacc):
  c = lax.axis_index("c")
  s = lax.axis_index("s")
  w = c * NS + s

  def zb(i, _):
    buf[pl.ds(i * 16, 16)] = jnp.zeros((16,), _f32)
    return 0
  lax.fori_loop(0, 640 // 16, zb, 0)
  pltpu.sync_copy(buf, dacc.at[pl.ds(s * 640, 640)])

  def ob(i, _):
    buf[pl.ds(i * 16, 16)] = jnp.ones((16,), _f32)
    return 0
  lax.fori_loop(0, EPR // 16, ob, 0)
  plsc.subcore_barrier()

  def outer(j, _):
    pltpu.sync_copy(dst_hbm.at[pl.ds(w * RPW + j * 8, 8)], didx)

    def body(k, _):
      pltpu.sync_copy(buf.at[pl.ds(0, EPR)], dacc.at[didx.at[k]], add=True)
      return 0
    lax.fori_loop(0, 8, body, 0)
    return 0
  lax.fori_loop(0, RPW // 8, outer, 0)
  plsc.subcore_barrier()
  pltpu.sync_copy(dacc.at[pl.ds(s * 640, 640)], out_hbm.at[c, pl.ds(s * 640, 640)])


@functools.partial(
    pl.kernel,
    out_type=jax.ShapeDtypeStruct((NC, NACC, D), _f32),
    mesh=_mesh,
    compiler_params=_sc_params,
    scratch_types=[
        pltpu.MemorySpace.VMEM((8, EPR), jnp.int32),
        pltpu.MemorySpace.VMEM((8, EPR), jnp.int32),
        pltpu.MemorySpace.VMEM((EPR, D), _f32),
        pltpu.MemorySpace.VMEM_SHARED((NACC, D), _f32),
        pltpu.SemaphoreType.DMA,
    ],
)
def _segsum_kernel(xs_hbm, src_hbm, dst_hbm, out_hbm, sidx, didx, rows, acc, sem):
  c = lax.axis_index("c")
  s = lax.axis_index("s")
  w = c * NS + s

  _zero_vmem_2d(rows, EPR)
  _zero_acc_slice(rows, acc, s)
  plsc.subcore_barrier()

  def outer(j, _):
    pltpu.sync_copy(src_hbm.at[pl.ds(w * RPW + j * 8, 8)], sidx)
    pltpu.sync_copy(dst_hbm.at[pl.ds(w * RPW + j * 8, 8)], didx)

    def body(k, _):
      pltpu.async_copy(xs_hbm.at[sidx.at[k]], rows, sem).wait()
      pltpu.sync_copy(rows, acc.at[didx.at[k]], add=True)
      return 0
    lax.fori_loop(0, 8, body, 0)
    return 0
  lax.fori_loop(0, RPW // 8, outer, 0)
  plsc.subcore_barrier()
  _dump_acc_slice(acc, out_hbm, c, s)


@functools.partial(
    pl.kernel,
    out_type=jax.ShapeDtypeStruct((NC, NACC, D), _f32),
    mesh=_mesh,
    compiler_params=_sc_params,
    scratch_types=[
        pltpu.MemorySpace.VMEM((8, EPR), jnp.int32),
        pltpu.MemorySpace.VMEM((8, EPR), jnp.int32),
        pltpu.MemorySpace.VMEM((EPR, D), _f32),
        pltpu.MemorySpace.VMEM((EPR, D), _f32),
        pltpu.MemorySpace.VMEM((16, 16), _f32),
        pltpu.MemorySpace.VMEM((16,), _f32),
        pltpu.MemorySpace.VMEM_SHARED((NACC, D), _f32),
        pltpu.SemaphoreType.DMA,
        pltpu.SemaphoreType.DMA,
    ],
)
def _gated_kernel(h_hbm, hl_hbm, src_hbm, dst_hbm, out_hbm,
                  sidx, didx, arows, brows, pbuf, wbuf, acc, sem_a, sem_b):
  c = lax.axis_index("c")
  s = lax.axis_index("s")
  w = c * NS + s

  _zero_vmem_2d(arows, EPR)
  _zero_acc_slice(arows, acc, s)
  plsc.subcore_barrier()

  il = jnp.arange(16, dtype=jnp.int32)

  def outer(j, _):
    pltpu.sync_copy(src_hbm.at[pl.ds(w * RPW + j * 8, 8)], sidx)
    pltpu.sync_copy(dst_hbm.at[pl.ds(w * RPW + j * 8, 8)], didx)
    lax.fori_loop(0, 8, _chunk, 0)
    return 0

  def _chunk(k, _):
    cp_a = pltpu.async_copy(h_hbm.at[sidx.at[k]], arows, sem_a)
    cp_b = pltpu.async_copy(hl_hbm.at[didx.at[k]], brows, sem_b)
    cp_a.wait()
    cp_b.wait()

    def grp(q, _):
      base = q * 16
      for r in range(16):
        e = base + r
        p = arows[e, pl.ds(0, 16)] * brows[e, pl.ds(0, 16)]
        for cc in range(1, D // 16):
          p = p + arows[e, pl.ds(cc * 16, 16)] * brows[e, pl.ds(cc * 16, 16)]
        plsc.store_scatter(pbuf, [il, jnp.full((16,), r, jnp.int32)], p)
      sv = pbuf[0, :]
      for l in range(1, 16):
        sv = sv + pbuf[l, :]
      wbuf[...] = 1.0 / (1.0 + jnp.exp(-sv))
      for r in range(16):
        e = base + r
        wv = plsc.load_gather(wbuf, [jnp.full((16,), r, jnp.int32)])
        for cc in range(D // 16):
          arows[e, pl.ds(cc * 16, 16)] = arows[e, pl.ds(cc * 16, 16)] * wv
      return 0
    lax.fori_loop(0, EPR // 16, grp, 0)
    pltpu.sync_copy(arows, acc.at[didx.at[k]], add=True)
    return 0
  lax.fori_loop(0, RPW // 8, outer, 0)
  plsc.subcore_barrier()
  _dump_acc_slice(acc, out_hbm, c, s)


def _tc1_body(deg_ref, x_ref, inv_ref, xs_ref):
  d = deg_ref[0] + deg_ref[1]
  inv = lax.rsqrt(jnp.maximum(d, 1.0))
  invn = inv[0:N, :]
  inv_ref[...] = invn
  xs_ref[0:N, :] = x_ref[...] * invn
  xs_ref[N:N_PAD, :] = jnp.zeros((N_PAD - N, D), _f32)


_tc1 = pl.pallas_call(
    _tc1_body,
    out_shape=(jax.ShapeDtypeStruct((N, 1), _f32),
               jax.ShapeDtypeStruct((N_PAD, D), _f32)),
)


def _tc2_body(p_ref, inv_ref, w_ref, o_ref):
  t = (p_ref[0, 0:N, :] + p_ref[1, 0:N, :]) * inv_ref[...]
  h = jnp.maximum(jnp.dot(t, w_ref[...], preferred_element_type=_f32), 0.0)
  o_ref[0:N, :] = h * inv_ref[...]
  o_ref[N:N_PAD, :] = jnp.zeros((N_PAD - N, D), _f32)


_tc2 = pl.pallas_call(
    _tc2_body,
    out_shape=jax.ShapeDtypeStruct((N_PAD, D), _f32),
)


def _tc3_body(p_ref, inv_ref, w_ref, wl_ref, h_ref, hl_ref):
  t = (p_ref[0, 0:N, :] + p_ref[1, 0:N, :]) * inv_ref[...]
  h = jnp.maximum(jnp.dot(t, w_ref[...], preferred_element_type=_f32), 0.0)
  h_ref[0:N, :] = h
  h_ref[N:N_PAD, :] = jnp.zeros((N_PAD - N, D), _f32)
  hl_ref[0:N, :] = h * wl_ref[...]
  hl_ref[N:N_PAD, :] = jnp.zeros((N_PAD - N, D), _f32)


_tc3 = pl.pallas_call(
    _tc3_body,
    out_shape=(jax.ShapeDtypeStruct((N_PAD, D), _f32),
               jax.ShapeDtypeStruct((N_PAD, D), _f32)),
)


def _tc4_body(p_ref, w_ref, o_ref):
  t = p_ref[0, 0:N, :] + p_ref[1, 0:N, :]
  o_ref[...] = jnp.maximum(jnp.dot(t, w_ref[...], preferred_element_type=_f32), 0.0)


_tc4 = pl.pallas_call(
    _tc4_body,
    out_shape=jax.ShapeDtypeStruct((N, D), _f32),
)


def kernel(x, edge_index, W1, W2, Wlink, Wnode):
  pad = jnp.full((E_PAD - E,), N, jnp.int32)
  src2 = jnp.concatenate([edge_index[0], pad]).reshape(NW * RPW, EPR)
  dst2 = jnp.concatenate([edge_index[1], pad]).reshape(NW * RPW, EPR)

  deg2 = _deg_kernel(dst2)
  inv, xs = _tc1(deg2.reshape(NC, DEG_PAD, 1), x)
  p1 = _segsum_kernel(xs, src2, dst2)
  xs2 = _tc2(p1, inv, W1)
  p2 = _segsum_kernel(xs2, src2, dst2)
  h, hl = _tc3(p2, inv, W2, Wlink.reshape(1, D))
  p3 = _gated_kernel(h, hl, src2, dst2)
  return _tc4(p3, Wnode)


# speedup vs baseline: 5.1292x; 1.0502x over previous
import functools

import jax
import jax.numpy as jnp
from jax import lax
from jax.experimental import pallas as pl
from jax.experimental.pallas import tpu as pltpu
from jax.experimental.pallas import tpu_sc as plsc

N = 10000
D = 128
E = 320000
NC = 2
NS = 16
NW = NC * NS
EPR = 128
RPW = 80
E_PAD = NW * RPW * EPR
N_PAD = N + 1
ROWS_PT = 632
NACC = ROWS_PT * NS
DEG_PAD = 640 * NS

_f32 = jnp.float32
_mesh = plsc.VectorSubcoreMesh(core_axis_name="c", subcore_axis_name="s")
_sc_params = pltpu.CompilerParams(needs_layout_passes=False)


def _zero_vmem_2d(ref, nrows):
  def zr(i, _):
    for cc in range(D // 16):
      ref[i, pl.ds(cc * 16, 16)] = jnp.zeros((16,), _f32)
    return 0
  lax.fori_loop(0, nrows, zr, 0)


def _zero_acc_slice(rows, acc, s):
  for j in range(4):
    pltpu.sync_copy(rows.at[pl.ds(0, EPR)],
                    acc.at[pl.ds(s * ROWS_PT + j * EPR, EPR)])
  pltpu.sync_copy(rows.at[pl.ds(0, ROWS_PT - 4 * EPR)],
                  acc.at[pl.ds(s * ROWS_PT + 4 * EPR, ROWS_PT - 4 * EPR)])


def _dump_acc_slice(acc, out_hbm, c, s):
  for j in range(4):
    pltpu.sync_copy(acc.at[pl.ds(s * ROWS_PT + j * EPR, EPR)],
                    out_hbm.at[c, pl.ds(s * ROWS_PT + j * EPR, EPR)])
  pltpu.sync_copy(acc.at[pl.ds(s * ROWS_PT + 4 * EPR, ROWS_PT - 4 * EPR)],
                  out_hbm.at[c, pl.ds(s * ROWS_PT + 4 * EPR, ROWS_PT - 4 * EPR)])


@functools.partial(
    pl.kernel,
    out_type=jax.ShapeDtypeStruct((NC, DEG_PAD), _f32),
    mesh=_mesh,
    compiler_params=_sc_params,
    scratch_types=[
        pltpu.MemorySpace.VMEM((RPW, EPR), jnp.int32),
        pltpu.MemorySpace.VMEM((640,), _f32),
        pltpu.MemorySpace.VMEM_SHARED((DEG_PAD,), _f32),
        pltpu.SemaphoreType.DMA,
    ],
)
def _deg_kernel(dst_hbm, out_hbm, didx, buf, dacc, ssem):
  c = lax.axis_index("c")
  s = lax.axis_index("s")
  w = c * NS + s

  def zb(i, _):
    buf[pl.ds(i * 16, 16)] = jnp.zeros((16,), _f32)
    return 0
  lax.fori_loop(0, 640 // 16, zb, 0)
  pltpu.sync_copy(buf, dacc.at[pl.ds(s * 640, 640)])
  pltpu.sync_copy(dst_hbm.at[pl.ds(w * RPW, RPW)], didx)

  def ob(i, _):
    buf[pl.ds(i * 16, 16)] = jnp.ones((16,), _f32)
    return 0
  lax.fori_loop(0, EPR // 16, ob, 0)
  plsc.subcore_barrier()

  def fire(g, _):
    pltpu.async_copy(buf.at[pl.ds(0, EPR)], dacc.at[didx.at[g]], ssem,
                     add=True)
    return 0
  lax.fori_loop(0, RPW, fire, 0)

  def drain(g, _):
    pltpu.make_async_copy(buf.at[pl.ds(0, EPR)], dacc.at[didx.at[0]],
                          ssem).wait()
    return 0
  lax.fori_loop(0, RPW, drain, 0)
  plsc.subcore_barrier()
  pltpu.sync_copy(dacc.at[pl.ds(s * 640, 640)], out_hbm.at[c, pl.ds(s * 640, 640)])


@functools.partial(
    pl.kernel,
    out_type=jax.ShapeDtypeStruct((NC, NACC, D), _f32),
    mesh=_mesh,
    compiler_params=_sc_params,
    scratch_types=[
        pltpu.MemorySpace.VMEM((16, EPR), jnp.int32),
        pltpu.MemorySpace.VMEM((16, EPR), jnp.int32),
        pltpu.MemorySpace.VMEM((2 * EPR, D), _f32),
        pltpu.MemorySpace.VMEM_SHARED((NACC, D), _f32),
        pltpu.SemaphoreType.DMA((2,)),
        pltpu.SemaphoreType.DMA((2,)),
    ],
)
def _segsum_kernel(xs_hbm, src_hbm, dst_hbm, out_hbm, sidx, didx, rows, acc,
                   gsem, isem):
  c = lax.axis_index("c")
  s = lax.axis_index("s")
  w = c * NS + s

  _zero_vmem_2d(rows, 2 * EPR)
  _zero_acc_slice(rows, acc, s)
  base = w * RPW
  pltpu.sync_copy(src_hbm.at[pl.ds(base, 8)], sidx.at[pl.ds(0, 8)])
  pltpu.sync_copy(dst_hbm.at[pl.ds(base, 8)], didx.at[pl.ds(0, 8)])
  plsc.subcore_barrier()
  pltpu.async_copy(xs_hbm.at[sidx.at[0]], rows.at[pl.ds(0, EPR)], gsem.at[0])

  def body(g, _):
    b = lax.rem(g, 2)
    j = lax.div(g, 8)
    k = lax.rem(g, 8)
    q = lax.rem(j, 2)
    pltpu.make_async_copy(
        xs_hbm.at[sidx.at[0]], rows.at[pl.ds(b * EPR, EPR)], gsem.at[b]
    ).wait()

    @pl.when(g + 1 < RPW)
    def _():
      g1 = g + 1
      b1 = lax.rem(g1, 2)
      k1 = lax.rem(g1, 8)
      q1 = lax.rem(lax.div(g1, 8), 2)

      @pl.when(k1 == 0)
      def _():
        pltpu.make_async_copy(
            src_hbm.at[pl.ds(base, 8)], sidx.at[pl.ds(0, 8)], isem.at[0]
        ).wait()
        pltpu.make_async_copy(
            dst_hbm.at[pl.ds(base, 8)], didx.at[pl.ds(0, 8)], isem.at[1]
        ).wait()
      pltpu.async_copy(
          xs_hbm.at[sidx.at[q1 * 8 + k1]],
          rows.at[pl.ds(b1 * EPR, EPR)],
          gsem.at[b1],
      )

    pltpu.sync_copy(rows.at[pl.ds(b * EPR, EPR)], acc.at[didx.at[q * 8 + k]],
                    add=True)

    @pl.when(jnp.logical_and(k == 0, g + 8 < RPW))
    def _():
      qn = 1 - q
      pltpu.async_copy(src_hbm.at[pl.ds(base + (j + 1) * 8, 8)],
                       sidx.at[pl.ds(qn * 8, 8)], isem.at[0])
      pltpu.async_copy(dst_hbm.at[pl.ds(base + (j + 1) * 8, 8)],
                       didx.at[pl.ds(qn * 8, 8)], isem.at[1])
    return 0
  lax.fori_loop(0, RPW, body, 0)
  plsc.subcore_barrier()
  _dump_acc_slice(acc, out_hbm, c, s)


@functools.partial(
    pl.kernel,
    out_type=jax.ShapeDtypeStruct((NC, NACC, D), _f32),
    mesh=_mesh,
    compiler_params=_sc_params,
    scratch_types=[
        pltpu.MemorySpace.VMEM((8, EPR), jnp.int32),
        pltpu.MemorySpace.VMEM((8, EPR), jnp.int32),
        pltpu.MemorySpace.VMEM((EPR, D), _f32),
        pltpu.MemorySpace.VMEM((EPR, D), _f32),
        pltpu.MemorySpace.VMEM((16, 16), _f32),
        pltpu.MemorySpace.VMEM((16,), _f32),
        pltpu.MemorySpace.VMEM_SHARED((NACC, D), _f32),
        pltpu.SemaphoreType.DMA,
        pltpu.SemaphoreType.DMA,
    ],
)
def _gated_kernel(h_hbm, hl_hbm, src_hbm, dst_hbm, out_hbm,
                  sidx, didx, arows, brows, pbuf, wbuf, acc, sem_a, sem_b):
  c = lax.axis_index("c")
  s = lax.axis_index("s")
  w = c * NS + s

  _zero_vmem_2d(arows, EPR)
  _zero_acc_slice(arows, acc, s)
  plsc.subcore_barrier()

  il = jnp.arange(16, dtype=jnp.int32)

  def outer(j, _):
    pltpu.sync_copy(src_hbm.at[pl.ds(w * RPW + j * 8, 8)], sidx)
    pltpu.sync_copy(dst_hbm.at[pl.ds(w * RPW + j * 8, 8)], didx)
    lax.fori_loop(0, 8, _chunk, 0)
    return 0

  def _chunk(k, _):
    cp_a = pltpu.async_copy(h_hbm.at[sidx.at[k]], arows, sem_a)
    cp_b = pltpu.async_copy(hl_hbm.at[didx.at[k]], brows, sem_b)
    cp_a.wait()
    cp_b.wait()

    def grp(q, _):
      base = q * 16
      for r in range(16):
        e = base + r
        p = arows[e, pl.ds(0, 16)] * brows[e, pl.ds(0, 16)]
        for cc in range(1, D // 16):
          p = p + arows[e, pl.ds(cc * 16, 16)] * brows[e, pl.ds(cc * 16, 16)]
        plsc.store_scatter(pbuf, [il, jnp.full((16,), r, jnp.int32)], p)
      sv = pbuf[0, :]
      for l in range(1, 16):
        sv = sv + pbuf[l, :]
      wbuf[...] = 1.0 / (1.0 + jnp.exp(-sv))
      for r in range(16):
        e = base + r
        wv = plsc.load_gather(wbuf, [jnp.full((16,), r, jnp.int32)])
        for cc in range(D // 16):
          arows[e, pl.ds(cc * 16, 16)] = arows[e, pl.ds(cc * 16, 16)] * wv
      return 0
    lax.fori_loop(0, EPR // 16, grp, 0)
    pltpu.sync_copy(arows, acc.at[didx.at[k]], add=True)
    return 0
  lax.fori_loop(0, RPW // 8, outer, 0)
  plsc.subcore_barrier()
  _dump_acc_slice(acc, out_hbm, c, s)


def _tc1_body(deg_ref, x_ref, inv_ref, xs_ref):
  d = deg_ref[0] + deg_ref[1]
  inv = lax.rsqrt(jnp.maximum(d, 1.0))
  invn = inv[0:N, :]
  inv_ref[...] = invn
  xs_ref[0:N, :] = x_ref[...] * invn
  xs_ref[N:N_PAD, :] = jnp.zeros((N_PAD - N, D), _f32)


_tc1 = pl.pallas_call(
    _tc1_body,
    out_shape=(jax.ShapeDtypeStruct((N, 1), _f32),
               jax.ShapeDtypeStruct((N_PAD, D), _f32)),
)


def _tc2_body(p_ref, inv_ref, w_ref, o_ref):
  t = (p_ref[0, 0:N, :] + p_ref[1, 0:N, :]) * inv_ref[...]
  h = jnp.maximum(jnp.dot(t, w_ref[...], preferred_element_type=_f32), 0.0)
  o_ref[0:N, :] = h * inv_ref[...]
  o_ref[N:N_PAD, :] = jnp.zeros((N_PAD - N, D), _f32)


_tc2 = pl.pallas_call(
    _tc2_body,
    out_shape=jax.ShapeDtypeStruct((N_PAD, D), _f32),
)


def _tc3_body(p_ref, inv_ref, w_ref, wl_ref, h_ref, hl_ref):
  t = (p_ref[0, 0:N, :] + p_ref[1, 0:N, :]) * inv_ref[...]
  h = jnp.maximum(jnp.dot(t, w_ref[...], preferred_element_type=_f32), 0.0)
  h_ref[0:N, :] = h
  h_ref[N:N_PAD, :] = jnp.zeros((N_PAD - N, D), _f32)
  hl_ref[0:N, :] = h * wl_ref[...]
  hl_ref[N:N_PAD, :] = jnp.zeros((N_PAD - N, D), _f32)


_tc3 = pl.pallas_call(
    _tc3_body,
    out_shape=(jax.ShapeDtypeStruct((N_PAD, D), _f32),
               jax.ShapeDtypeStruct((N_PAD, D), _f32)),
)


def _tc4_body(p_ref, w_ref, o_ref):
  t = p_ref[0, 0:N, :] + p_ref[1, 0:N, :]
  o_ref[...] = jnp.maximum(jnp.dot(t, w_ref[...], preferred_element_type=_f32), 0.0)


_tc4 = pl.pallas_call(
    _tc4_body,
    out_shape=jax.ShapeDtypeStruct((N, D), _f32),
)


def kernel(x, edge_index, W1, W2, Wlink, Wnode):
  pad = jnp.full((E_PAD - E,), N, jnp.int32)
  src2 = jnp.concatenate([edge_index[0], pad]).reshape(NW * RPW, EPR)
  dst2 = jnp.concatenate([edge_index[1], pad]).reshape(NW * RPW, EPR)

  deg2 = _deg_kernel(dst2)
  inv, xs = _tc1(deg2.reshape(NC, DEG_PAD, 1), x)
  p1 = _segsum_kernel(xs, src2, dst2)
  xs2 = _tc2(p1, inv, W1)
  p2 = _segsum_kernel(xs2, src2, dst2)
  h, hl = _tc3(p2, inv, W2, Wlink.reshape(1, D))
  p3 = _gated_kernel(h, hl, src2, dst2)
  return _tc4(p3, Wnode)
